# baseline (speedup 1.0000x reference)
import functools

import jax
import jax.numpy as jnp
from jax import lax
from jax.experimental import pallas as pl
from jax.experimental.pallas import tpu as pltpu
from jax.experimental.pallas import tpu_sc as plsc

VOCAB = 100000
EMBED = 128
CTX = 20
HID = 128
NB = 25
TILE = VOCAB // NB
IDX_PAD = 32


def _sc_gather(idx_pad, table):
    mesh = plsc.VectorSubcoreMesh(core_axis_name="c", subcore_axis_name="s")

    @functools.partial(
        pl.kernel,
        mesh=mesh,
        out_type=jax.ShapeDtypeStruct((IDX_PAD, EMBED), jnp.float32),
        scratch_types=[
            pltpu.VMEM((IDX_PAD,), jnp.int32),
            pltpu.VMEM((IDX_PAD, EMBED), jnp.float32),
            pltpu.SemaphoreType.DMA,
        ],
    )
    def gather_kernel(idx_hbm, table_hbm, out_hbm, idx_v, rows_v, sem):
        wid = lax.axis_index("s") * 2 + lax.axis_index("c")

        @pl.when(wid == 0)
        def _():
            pltpu.sync_copy(idx_hbm, idx_v)
            pltpu.async_copy(table_hbm.at[idx_v], rows_v, sem).wait()
            pltpu.sync_copy(rows_v, out_hbm)

    return gather_kernel(idx_pad, table)


def _main_body(flat_ref, w1_ref, b1_ref, w2_ref, b2_ref,
               out_ref, stats_ref, h_s, m_s, s_s):
    i = pl.program_id(0)

    @pl.when(i == 0)
    def _():
        h = lax.dot_general(flat_ref[...], w1_ref[...],
                            (((1,), (1,)), ((), ())),
                            preferred_element_type=jnp.float32)
        h_s[...] = jnp.maximum(h + b1_ref[...], 0.0)
        m_s[0] = -1e30
        s_s[0] = 0.0

    tile = lax.dot_general(h_s[...], w2_ref[...],
                           (((1,), (1,)), ((), ())),
                           preferred_element_type=jnp.float32)
    tile = tile + b2_ref[0]
    out_ref[0] = tile

    m_old = m_s[0]
    new_m = jnp.maximum(m_old, jnp.max(tile))
    s_s[0] = s_s[0] * jnp.exp(m_old - new_m) + jnp.sum(jnp.exp(tile - new_m))
    m_s[0] = new_m

    @pl.when(i == NB - 1)
    def _():
        stats_ref[...] = jnp.full((1, 128), m_s[0] + jnp.log(s_s[0]),
                                  jnp.float32)


def _norm_body(logits_ref, stats_ref, out_ref):
    out_ref[0] = logits_ref[0] - stats_ref[0:1, 0:1]


def kernel(inputs, emb, W1, b1, W2, b2):
    idx = inputs.astype(jnp.int32)
    idx_pad = jnp.concatenate(
        [idx, jnp.zeros((IDX_PAD - CTX,), jnp.int32)])

    rows = _sc_gather(idx_pad, emb)
    flat = rows[:CTX].reshape(1, CTX * EMBED)

    logits, stats = pl.pallas_call(
        _main_body,
        grid=(NB,),
        in_specs=[
            pl.BlockSpec((1, CTX * EMBED), lambda i: (0, 0)),
            pl.BlockSpec((HID, CTX * EMBED), lambda i: (0, 0)),
            pl.BlockSpec((1, HID), lambda i: (0, 0)),
            pl.BlockSpec((TILE, HID), lambda i: (i, 0)),
            pl.BlockSpec((1, 1, TILE), lambda i: (i, 0, 0)),
        ],
        out_specs=[
            pl.BlockSpec((1, 1, TILE), lambda i: (i, 0, 0)),
            pl.BlockSpec((1, 128), lambda i: (0, 0)),
        ],
        out_shape=[
            jax.ShapeDtypeStruct((NB, 1, TILE), jnp.float32),
            jax.ShapeDtypeStruct((1, 128), jnp.float32),
        ],
        scratch_shapes=[
            pltpu.VMEM((1, HID), jnp.float32),
            pltpu.SMEM((1,), jnp.float32),
            pltpu.SMEM((1,), jnp.float32),
        ],
    )(flat, W1, b1.reshape(1, HID), W2, b2.reshape(NB, 1, TILE))

    log_probs = pl.pallas_call(
        _norm_body,
        grid=(NB,),
        in_specs=[
            pl.BlockSpec((1, 1, TILE), lambda i: (i, 0, 0)),
            pl.BlockSpec((1, 128), lambda i: (0, 0)),
        ],
        out_specs=pl.BlockSpec((1, 1, TILE), lambda i: (i, 0, 0)),
        out_shape=jax.ShapeDtypeStruct((NB, 1, TILE), jnp.float32),
    )(logits, stats)

    return log_probs.reshape(1, VOCAB)


# speedup vs baseline: 1.0044x; 1.0044x over previous
import functools

import jax
import jax.numpy as jnp
from jax import lax
from jax.experimental import pallas as pl
from jax.experimental.pallas import tpu as pltpu
from jax.experimental.pallas import tpu_sc as plsc

VOCAB = 100000
EMBED = 128
CTX = 20
HID = 128
NB = 25
TILE = VOCAB // NB
IDX_PAD = 32


def _sc_gather(idx_pad, table):
    mesh = plsc.VectorSubcoreMesh(core_axis_name="c", subcore_axis_name="s")

    @functools.partial(
        pl.kernel,
        mesh=mesh,
        out_type=jax.ShapeDtypeStruct((IDX_PAD, EMBED), jnp.float32),
        scratch_types=[
            pltpu.VMEM((IDX_PAD,), jnp.int32),
            pltpu.VMEM((IDX_PAD, EMBED), jnp.float32),
            pltpu.SemaphoreType.DMA,
        ],
    )
    def gather_kernel(idx_hbm, table_hbm, out_hbm, idx_v, rows_v, sem):
        wid = lax.axis_index("s") * 2 + lax.axis_index("c")

        @pl.when(wid == 0)
        def _():
            pltpu.sync_copy(idx_hbm, idx_v)
            pltpu.async_copy(table_hbm.at[idx_v], rows_v, sem).wait()
            pltpu.sync_copy(rows_v, out_hbm)

    return gather_kernel(idx_pad, table)


def _main_body(flat_ref, w1_ref, b1_ref, w2_ref, b2_ref,
               out_ref, stats_ref, h_s, m_s, s_s):
    i = pl.program_id(0)

    @pl.when(i == 0)
    def _():
        h = lax.dot_general(flat_ref[...], w1_ref[...],
                            (((1,), (1,)), ((), ())),
                            preferred_element_type=jnp.float32)
        h_s[...] = jnp.maximum(h + b1_ref[...], 0.0)
        m_s[0] = -1e30
        s_s[0] = 0.0

    tile = lax.dot_general(h_s[...].astype(jnp.bfloat16),
                           w2_ref[...].astype(jnp.bfloat16),
                           (((1,), (1,)), ((), ())),
                           preferred_element_type=jnp.float32)
    tile = tile + b2_ref[0]
    out_ref[0] = tile

    m_old = m_s[0]
    new_m = jnp.maximum(m_old, jnp.max(tile))
    s_s[0] = s_s[0] * jnp.exp(m_old - new_m) + jnp.sum(jnp.exp(tile - new_m))
    m_s[0] = new_m

    @pl.when(i == NB - 1)
    def _():
        stats_ref[...] = jnp.full((1, 128), m_s[0] + jnp.log(s_s[0]),
                                  jnp.float32)


def _norm_body(logits_ref, stats_ref, out_ref):
    out_ref[0] = logits_ref[0] - stats_ref[0:1, 0:1]


def kernel(inputs, emb, W1, b1, W2, b2):
    idx = inputs.astype(jnp.int32)
    idx_pad = jnp.concatenate(
        [idx, jnp.zeros((IDX_PAD - CTX,), jnp.int32)])

    rows = _sc_gather(idx_pad, emb)
    flat = rows[:CTX].reshape(1, CTX * EMBED)

    logits, stats = pl.pallas_call(
        _main_body,
        grid=(NB,),
        in_specs=[
            pl.BlockSpec((1, CTX * EMBED), lambda i: (0, 0)),
            pl.BlockSpec((HID, CTX * EMBED), lambda i: (0, 0)),
            pl.BlockSpec((1, HID), lambda i: (0, 0)),
            pl.BlockSpec((TILE, HID), lambda i: (i, 0)),
            pl.BlockSpec((1, 1, TILE), lambda i: (i, 0, 0)),
        ],
        out_specs=[
            pl.BlockSpec((1, 1, TILE), lambda i: (i, 0, 0)),
            pl.BlockSpec((1, 128), lambda i: (0, 0)),
        ],
        out_shape=[
            jax.ShapeDtypeStruct((NB, 1, TILE), jnp.float32),
            jax.ShapeDtypeStruct((1, 128), jnp.float32),
        ],
        scratch_shapes=[
            pltpu.VMEM((1, HID), jnp.float32),
            pltpu.SMEM((1,), jnp.float32),
            pltpu.SMEM((1,), jnp.float32),
        ],
    )(flat, W1, b1.reshape(1, HID), W2, b2.reshape(NB, 1, TILE))

    log_probs = pl.pallas_call(
        _norm_body,
        grid=(NB,),
        in_specs=[
            pl.BlockSpec((1, 1, TILE), lambda i: (i, 0, 0)),
            pl.BlockSpec((1, 128), lambda i: (0, 0)),
        ],
        out_specs=pl.BlockSpec((1, 1, TILE), lambda i: (i, 0, 0)),
        out_shape=jax.ShapeDtypeStruct((NB, 1, TILE), jnp.float32),
    )(logits, stats)

    return log_probs.reshape(1, VOCAB)


# speedup vs baseline: 1.9196x; 1.9111x over previous
import jax
import jax.numpy as jnp
from jax import lax
from jax.experimental import pallas as pl
from jax.experimental.pallas import tpu as pltpu

VOCAB = 100000
EMBED = 128
CTX = 20
HID = 128
NB = 25
TILE = VOCAB // NB


def _body(idx_ref, emb_ref, w1_ref, b1_ref, w2_ref, b2_ref, out_ref,
          gbuf, h_s, m_s, s_s, sem):
    i = pl.program_id(0)

    @pl.when(i == 0)
    def _():
        copies = [
            pltpu.make_async_copy(
                emb_ref.at[pl.ds(idx_ref[t], 1), :],
                gbuf.at[pl.ds(t, 1), :],
                sem,
            )
            for t in range(CTX)
        ]
        for c in copies:
            c.start()
        for c in copies:
            c.wait()
        acc = jnp.zeros((1, HID), jnp.float32)
        for t in range(CTX):
            acc += lax.dot_general(
                gbuf[t:t + 1, :],
                w1_ref[:, t * EMBED:(t + 1) * EMBED],
                (((1,), (1,)), ((), ())),
                preferred_element_type=jnp.float32)
        h_s[...] = jnp.maximum(acc + b1_ref[...], 0.0)
        m_s[0] = -1e30
        s_s[0] = 0.0

    @pl.when(i < NB)
    def _():
        tile = lax.dot_general(h_s[...], w2_ref[...],
                               (((1,), (1,)), ((), ())),
                               preferred_element_type=jnp.float32)
        tile = tile + b2_ref[0]
        out_ref[i] = tile

        m_old = m_s[0]
        new_m = jnp.maximum(m_old, jnp.max(tile))
        s_s[0] = (s_s[0] * jnp.exp(m_old - new_m)
                  + jnp.sum(jnp.exp(tile - new_m)))
        m_s[0] = new_m

    @pl.when(i == NB)
    def _():
        out_ref[...] = out_ref[...] - (m_s[0] + jnp.log(s_s[0]))


def kernel(inputs, emb, W1, b1, W2, b2):
    idx = inputs.astype(jnp.int32)

    grid_spec = pltpu.PrefetchScalarGridSpec(
        num_scalar_prefetch=1,
        grid=(NB + 1,),
        in_specs=[
            pl.BlockSpec(memory_space=pltpu.HBM),
            pl.BlockSpec((HID, CTX * EMBED), lambda i, s: (0, 0)),
            pl.BlockSpec((1, HID), lambda i, s: (0, 0)),
            pl.BlockSpec((TILE, HID),
                         lambda i, s: (jnp.minimum(i, NB - 1), 0)),
            pl.BlockSpec((1, 1, TILE),
                         lambda i, s: (jnp.minimum(i, NB - 1), 0, 0)),
        ],
        out_specs=pl.BlockSpec((NB, 1, TILE), lambda i, s: (0, 0, 0)),
        scratch_shapes=[
            pltpu.VMEM((CTX, EMBED), jnp.float32),
            pltpu.VMEM((1, HID), jnp.float32),
            pltpu.SMEM((1,), jnp.float32),
            pltpu.SMEM((1,), jnp.float32),
            pltpu.SemaphoreType.DMA,
        ],
    )

    log_probs = pl.pallas_call(
        _body,
        grid_spec=grid_spec,
        out_shape=jax.ShapeDtypeStruct((NB, 1, TILE), jnp.float32),
    )(idx, emb, W1, b1.reshape(1, HID), W2, b2.reshape(NB, 1, TILE))

    return log_probs.reshape(1, VOCAB)


# speedup vs baseline: 1.9305x; 1.0057x over previous
import jax
import jax.numpy as jnp
from jax import lax
from jax.experimental import pallas as pl
from jax.experimental.pallas import tpu as pltpu

VOCAB = 100000
EMBED = 128
CTX = 20
HID = 128
NB = 25
TILE = VOCAB // NB


def _body(idx_ref, emb_ref, w1_ref, b1_ref, w2_ref, b2_ref, out_ref,
          gbuf, h_s, sem):
    i = pl.program_id(0)

    @pl.when(i == 0)
    def _():
        copies = [
            pltpu.make_async_copy(
                emb_ref.at[pl.ds(idx_ref[t], 1), :],
                gbuf.at[pl.ds(t, 1), :],
                sem,
            )
            for t in range(CTX)
        ]
        for c in copies:
            c.start()
        for c in copies:
            c.wait()
        acc = jnp.zeros((1, HID), jnp.float32)
        for t in range(CTX):
            acc += lax.dot_general(
                gbuf[t:t + 1, :],
                w1_ref[:, t * EMBED:(t + 1) * EMBED],
                (((1,), (1,)), ((), ())),
                preferred_element_type=jnp.float32)
        h_s[...] = jnp.maximum(acc + b1_ref[...], 0.0)

    @pl.when(i < NB)
    def _():
        tile = lax.dot_general(h_s[...], w2_ref[...],
                               (((1,), (1,)), ((), ())),
                               preferred_element_type=jnp.float32)
        out_ref[i] = tile + b2_ref[0]

    @pl.when(i == NB)
    def _():
        allv = out_ref[...]
        m = jnp.max(allv)
        s = jnp.sum(jnp.exp(allv - m))
        out_ref[...] = allv - (m + jnp.log(s))


def kernel(inputs, emb, W1, b1, W2, b2):
    idx = inputs.astype(jnp.int32)

    grid_spec = pltpu.PrefetchScalarGridSpec(
        num_scalar_prefetch=1,
        grid=(NB + 1,),
        in_specs=[
            pl.BlockSpec(memory_space=pltpu.HBM),
            pl.BlockSpec((HID, CTX * EMBED), lambda i, s: (0, 0)),
            pl.BlockSpec((1, HID), lambda i, s: (0, 0)),
            pl.BlockSpec((TILE, HID),
                         lambda i, s: (jnp.minimum(i, NB - 1), 0)),
            pl.BlockSpec((1, 1, TILE),
                         lambda i, s: (jnp.minimum(i, NB - 1), 0, 0)),
        ],
        out_specs=pl.BlockSpec((NB, 1, TILE), lambda i, s: (0, 0, 0)),
        scratch_shapes=[
            pltpu.VMEM((CTX, EMBED), jnp.float32),
            pltpu.VMEM((1, HID), jnp.float32),
            pltpu.SemaphoreType.DMA,
        ],
    )

    log_probs = pl.pallas_call(
        _body,
        grid_spec=grid_spec,
        out_shape=jax.ShapeDtypeStruct((NB, 1, TILE), jnp.float32),
    )(idx, emb, W1, b1.reshape(1, HID), W2, b2.reshape(NB, 1, TILE))

    return log_probs.reshape(1, VOCAB)
